# Initial kernel scaffold; baseline (speedup 1.0000x reference)
import jax
import jax.numpy as jnp
from jax.experimental import pallas as pl


def kernel(x, edge_index, W1, b1, W2, b2, Wl, bl):
    raise NotImplementedError("write your pallas kernel here")


# speedup vs baseline: 14.2999x; 14.2999x over previous
import functools
import jax
import jax.numpy as jnp
from jax import lax
from jax.experimental import pallas as pl
from jax.experimental.pallas import tpu as pltpu
from jax.experimental.pallas import tpu_sc as plsc

NC = 2
NS = 16
CH = 100
SLAB_I = 40


def _sc_mesh():
    return plsc.VectorSubcoreMesh(
        core_axis_name="c", subcore_axis_name="s", num_cores=NC, num_subcores=NS
    )


def _make_deg_kernel(n, e):
    cpt = e // (NS * CH)
    nsl = cpt // SLAB_I

    @functools.partial(
        pl.kernel,
        out_type=jax.ShapeDtypeStruct((n,), jnp.float32),
        mesh=_sc_mesh(),
        scratch_types=[
            pltpu.VMEM_SHARED((n,), jnp.float32),
            pltpu.VMEM((SLAB_I, CH), jnp.int32),
            pltpu.VMEM((CH,), jnp.float32),
        ],
    )
    def deg_kernel(dst3d, zeros_hbm, ones_hbm, deg_out, acc, dstv, onesv):
        c = lax.axis_index("c")
        s = lax.axis_index("s")

        @pl.when(c == 0)
        def _():
            pltpu.sync_copy(ones_hbm, onesv)

            @pl.when(s == 0)
            def _():
                pltpu.sync_copy(zeros_hbm, acc)

            plsc.subcore_barrier()

            def slab_body(t, carry):
                pltpu.sync_copy(dst3d.at[s].at[pl.ds(t * SLAB_I, SLAB_I)], dstv)

                def chunk(j, carry2):
                    pltpu.sync_copy(onesv, acc.at[dstv.at[j]], add=True)
                    return carry2

                return lax.fori_loop(0, SLAB_I, chunk, carry)

            lax.fori_loop(0, nsl, slab_body, 0)
            plsc.subcore_barrier()

            @pl.when(s == 0)
            def _():
                pltpu.sync_copy(acc, deg_out)

    return deg_kernel


def _make_agg_kernel(n, e, d):
    cpt = e // (NS * CH)
    nsl = cpt // SLAB_I
    slab = (n // NS) // 8 * 8
    tail = n - slab * NS

    @functools.partial(
        pl.kernel,
        out_type=(
            jax.ShapeDtypeStruct((n, d), jnp.float32),
            jax.ShapeDtypeStruct((n, d), jnp.float32),
        ),
        mesh=_sc_mesh(),
        scratch_types=[
            pltpu.VMEM_SHARED((n, d), jnp.float32),
            pltpu.VMEM((SLAB_I, CH), jnp.int32),
            pltpu.VMEM((SLAB_I, CH), jnp.int32),
            pltpu.VMEM((CH, d), jnp.float32),
        ],
    )
    def agg_kernel(y0, y1, src3d, dst3d, o0, o1, acc, srcv, dstv, rows):
        c = lax.axis_index("c")
        s = lax.axis_index("s")

        def half(y_hbm, out_hbm):
            pltpu.sync_copy(y_hbm.at[pl.ds(s * slab, slab)], acc.at[pl.ds(s * slab, slab)])

            @pl.when(s == NS - 1)
            def _():
                pltpu.sync_copy(y_hbm.at[pl.ds(slab * NS, tail)], acc.at[pl.ds(slab * NS, tail)])

            plsc.subcore_barrier()

            def slab_body(t, carry):
                pltpu.sync_copy(src3d.at[s].at[pl.ds(t * SLAB_I, SLAB_I)], srcv)
                pltpu.sync_copy(dst3d.at[s].at[pl.ds(t * SLAB_I, SLAB_I)], dstv)

                def chunk(j, carry2):
                    pltpu.sync_copy(y_hbm.at[srcv.at[j]], rows)
                    pltpu.sync_copy(rows, acc.at[dstv.at[j]], add=True)
                    return carry2

                return lax.fori_loop(0, SLAB_I, chunk, carry)

            lax.fori_loop(0, nsl, slab_body, 0)
            plsc.subcore_barrier()
            pltpu.sync_copy(acc.at[pl.ds(s * slab, slab)], out_hbm.at[pl.ds(s * slab, slab)])

            @pl.when(s == NS - 1)
            def _():
                pltpu.sync_copy(acc.at[pl.ds(slab * NS, tail)], out_hbm.at[pl.ds(slab * NS, tail)])

        @pl.when(c == 0)
        def _():
            half(y0, o0)

        @pl.when(c == 1)
        def _():
            half(y1, o1)

    return agg_kernel


def _stage1_body(deg_ref, x_ref, w1t_ref, y0_ref, y1_ref, dis_ref):
    dis = lax.rsqrt(deg_ref[...] + 1.0)
    xl = jnp.dot(x_ref[...], w1t_ref[...], preferred_element_type=jnp.float32)
    y = xl * dis
    y0_ref[...] = y[:, :128]
    y1_ref[...] = y[:, 128:]
    dis_ref[...] = dis


def _stage2_body(a0_ref, a1_ref, dis_ref, b_ref, w2t_ref, z0_ref, z1_ref):
    dis = dis_ref[...]
    h0 = jnp.maximum(a0_ref[...] * dis + b_ref[0, :128], 0.0)
    h1 = jnp.maximum(a1_ref[...] * dis + b_ref[0, 128:], 0.0)
    xl = jnp.dot(h0, w2t_ref[0:128, :], preferred_element_type=jnp.float32)
    xl = xl + jnp.dot(h1, w2t_ref[128:256, :], preferred_element_type=jnp.float32)
    y = xl * dis
    z0_ref[...] = y[:, :128]
    z1_ref[...] = y[:, 128:]


def _stage3_body(a0_ref, a1_ref, dis_ref, b_ref, wlt_ref, bl_ref, out_ref):
    dis = dis_ref[...]
    h0 = jnp.maximum(a0_ref[...] * dis + b_ref[0, :128], 0.0)
    h1 = jnp.maximum(a1_ref[...] * dis + b_ref[0, 128:], 0.0)
    o = jnp.dot(h0, wlt_ref[0:128, :], preferred_element_type=jnp.float32)
    o = o + jnp.dot(h1, wlt_ref[128:256, :], preferred_element_type=jnp.float32)
    out_ref[...] = o + bl_ref[0, :]


def kernel(x, edge_index, W1, b1, W2, b2, Wl, bl):
    n, d_in = x.shape
    e = edge_index.shape[1]
    hid = W1.shape[0]
    ncls = Wl.shape[0]
    assert hid == 256 and d_in == 128 and n % NS == 0
    assert e % (NS * CH * SLAB_I) == 0

    cpt = e // (NS * CH)
    src3d = edge_index[0].reshape(NS, cpt, CH).astype(jnp.int32)
    dst3d = edge_index[1].reshape(NS, cpt, CH).astype(jnp.int32)
    zeros_n = jnp.zeros((n,), jnp.float32)
    ones_ch = jnp.ones((CH,), jnp.float32)

    deg = _make_deg_kernel(n, e)(dst3d, zeros_n, ones_ch).reshape(n, 1)

    blk = 1000
    grid = (n // blk,)

    y0, y1, dis = pl.pallas_call(
        _stage1_body,
        grid=grid,
        in_specs=[
            pl.BlockSpec((blk, 1), lambda i: (i, 0)),
            pl.BlockSpec((blk, d_in), lambda i: (i, 0)),
            pl.BlockSpec((d_in, hid), lambda i: (0, 0)),
        ],
        out_specs=[
            pl.BlockSpec((blk, 128), lambda i: (i, 0)),
            pl.BlockSpec((blk, 128), lambda i: (i, 0)),
            pl.BlockSpec((blk, 1), lambda i: (i, 0)),
        ],
        out_shape=[
            jax.ShapeDtypeStruct((n, 128), jnp.float32),
            jax.ShapeDtypeStruct((n, 128), jnp.float32),
            jax.ShapeDtypeStruct((n, 1), jnp.float32),
        ],
    )(deg, x, W1.T)

    agg = _make_agg_kernel(n, e, 128)
    a0, a1 = agg(y0, y1, src3d, dst3d)

    z0, z1 = pl.pallas_call(
        _stage2_body,
        grid=grid,
        in_specs=[
            pl.BlockSpec((blk, 128), lambda i: (i, 0)),
            pl.BlockSpec((blk, 128), lambda i: (i, 0)),
            pl.BlockSpec((blk, 1), lambda i: (i, 0)),
            pl.BlockSpec((1, hid), lambda i: (0, 0)),
            pl.BlockSpec((hid, hid), lambda i: (0, 0)),
        ],
        out_specs=[
            pl.BlockSpec((blk, 128), lambda i: (i, 0)),
            pl.BlockSpec((blk, 128), lambda i: (i, 0)),
        ],
        out_shape=[
            jax.ShapeDtypeStruct((n, 128), jnp.float32),
            jax.ShapeDtypeStruct((n, 128), jnp.float32),
        ],
    )(a0, a1, dis, b1.reshape(1, hid), W2.T)

    p0, p1 = agg(z0, z1, src3d, dst3d)

    out = pl.pallas_call(
        _stage3_body,
        grid=grid,
        in_specs=[
            pl.BlockSpec((blk, 128), lambda i: (i, 0)),
            pl.BlockSpec((blk, 128), lambda i: (i, 0)),
            pl.BlockSpec((blk, 1), lambda i: (i, 0)),
            pl.BlockSpec((1, hid), lambda i: (0, 0)),
            pl.BlockSpec((hid, ncls), lambda i: (0, 0)),
            pl.BlockSpec((1, ncls), lambda i: (0, 0)),
        ],
        out_specs=pl.BlockSpec((blk, ncls), lambda i: (i, 0)),
        out_shape=jax.ShapeDtypeStruct((n, ncls), jnp.float32),
    )(p0, p1, dis, b2.reshape(1, hid), Wl.T, bl.reshape(1, ncls))

    return out


# speedup vs baseline: 22.3537x; 1.5632x over previous
import functools
import jax
import jax.numpy as jnp
from jax import lax
from jax.experimental import pallas as pl
from jax.experimental.pallas import tpu as pltpu
from jax.experimental.pallas import tpu_sc as plsc

NC = 2
NS = 16
CH = 100
SLAB_I = 40
NB = 2


def _sc_mesh():
    return plsc.VectorSubcoreMesh(
        core_axis_name="c", subcore_axis_name="s", num_cores=NC, num_subcores=NS
    )


def _make_deg_kernel(n, e):
    cpt = e // (NS * CH)
    nsl = cpt // SLAB_I

    @functools.partial(
        pl.kernel,
        out_type=jax.ShapeDtypeStruct((n,), jnp.float32),
        mesh=_sc_mesh(),
        scratch_types=[
            pltpu.VMEM_SHARED((n,), jnp.float32),
            pltpu.VMEM((SLAB_I, CH), jnp.int32),
            pltpu.VMEM((CH,), jnp.float32),
        ],
    )
    def deg_kernel(dst3d, zeros_hbm, ones_hbm, deg_out, acc, dstv, onesv):
        c = lax.axis_index("c")
        s = lax.axis_index("s")

        @pl.when(c == 0)
        def _():
            pltpu.sync_copy(ones_hbm, onesv)

            @pl.when(s == 0)
            def _():
                pltpu.sync_copy(zeros_hbm, acc)

            plsc.subcore_barrier()

            def slab_body(t, carry):
                pltpu.sync_copy(dst3d.at[s].at[pl.ds(t * SLAB_I, SLAB_I)], dstv)

                def chunk(j, carry2):
                    pltpu.sync_copy(onesv, acc.at[dstv.at[j]], add=True)
                    return carry2

                return lax.fori_loop(0, SLAB_I, chunk, carry)

            lax.fori_loop(0, nsl, slab_body, 0)
            plsc.subcore_barrier()

            @pl.when(s == 0)
            def _():
                pltpu.sync_copy(acc, deg_out)

    return deg_kernel


def _make_agg_kernel(n, e, d):
    cpt = e // (NS * CH)
    nsl = cpt // SLAB_I
    slab = (n // NS) // 8 * 8
    tail = n - slab * NS

    @functools.partial(
        pl.kernel,
        out_type=(
            jax.ShapeDtypeStruct((n, d), jnp.float32),
            jax.ShapeDtypeStruct((n, d), jnp.float32),
        ),
        mesh=_sc_mesh(),
        scratch_types=[
            pltpu.VMEM_SHARED((n, d), jnp.float32),
            pltpu.VMEM((SLAB_I, CH), jnp.int32),
            pltpu.VMEM((SLAB_I, CH), jnp.int32),
            pltpu.VMEM((NB, CH, d), jnp.float32),
            pltpu.SemaphoreType.DMA,
            pltpu.SemaphoreType.DMA,
            pltpu.SemaphoreType.DMA,
            pltpu.SemaphoreType.DMA,
        ],
    )
    def agg_kernel(y0, y1, src3d, dst3d, o0, o1, acc, srcv, dstv, rows,
                   g0, g1, s0, s1):
        c = lax.axis_index("c")
        s = lax.axis_index("s")
        sem_g = [g0, g1]
        sem_s = [s0, s1]

        def half(y_hbm, out_hbm):
            pltpu.sync_copy(y_hbm.at[pl.ds(s * slab, slab)], acc.at[pl.ds(s * slab, slab)])

            @pl.when(s == NS - 1)
            def _():
                pltpu.sync_copy(y_hbm.at[pl.ds(slab * NS, tail)], acc.at[pl.ds(slab * NS, tail)])

            plsc.subcore_barrier()

            def slab_body(t, carry):
                pltpu.sync_copy(src3d.at[s].at[pl.ds(t * SLAB_I, SLAB_I)], srcv)
                pltpu.sync_copy(dst3d.at[s].at[pl.ds(t * SLAB_I, SLAB_I)], dstv)
                pltpu.async_copy(y_hbm.at[srcv.at[0]], rows.at[0], sem_g[0])

                def group(g, carry2):
                    for b in range(NB):
                        j = g * NB + b
                        bn = (b + 1) % NB
                        jn = j + 1

                        @pl.when(jn < SLAB_I)
                        def _():
                            @pl.when(jn >= NB)
                            def _():
                                pltpu.make_async_copy(
                                    rows.at[bn], acc.at[dstv.at[jn - NB]], sem_s[bn]
                                ).wait()

                            pltpu.async_copy(y_hbm.at[srcv.at[jn]], rows.at[bn], sem_g[bn])

                        pltpu.make_async_copy(
                            y_hbm.at[srcv.at[j]], rows.at[b], sem_g[b]
                        ).wait()
                        pltpu.async_copy(rows.at[b], acc.at[dstv.at[j]], sem_s[b], add=True)
                    return carry2

                lax.fori_loop(0, SLAB_I // NB, group, carry)
                for k in range(NB):
                    j = SLAB_I - NB + k
                    pltpu.make_async_copy(
                        rows.at[j % NB], acc.at[dstv.at[j]], sem_s[j % NB]
                    ).wait()
                return carry

            lax.fori_loop(0, nsl, slab_body, 0)
            plsc.subcore_barrier()
            pltpu.sync_copy(acc.at[pl.ds(s * slab, slab)], out_hbm.at[pl.ds(s * slab, slab)])

            @pl.when(s == NS - 1)
            def _():
                pltpu.sync_copy(acc.at[pl.ds(slab * NS, tail)], out_hbm.at[pl.ds(slab * NS, tail)])

        @pl.when(c == 0)
        def _():
            half(y0, o0)

        @pl.when(c == 1)
        def _():
            half(y1, o1)

    return agg_kernel


def _stage1_body(deg_ref, x_ref, w1t_ref, y0_ref, y1_ref, dis_ref):
    dis = lax.rsqrt(deg_ref[...] + 1.0)
    xl = jnp.dot(x_ref[...], w1t_ref[...], preferred_element_type=jnp.float32)
    y = xl * dis
    y0_ref[...] = y[:, :128]
    y1_ref[...] = y[:, 128:]
    dis_ref[...] = dis


def _stage2_body(a0_ref, a1_ref, dis_ref, b_ref, w2t_ref, z0_ref, z1_ref):
    dis = dis_ref[...]
    h0 = jnp.maximum(a0_ref[...] * dis + b_ref[0, :128], 0.0)
    h1 = jnp.maximum(a1_ref[...] * dis + b_ref[0, 128:], 0.0)
    xl = jnp.dot(h0, w2t_ref[0:128, :], preferred_element_type=jnp.float32)
    xl = xl + jnp.dot(h1, w2t_ref[128:256, :], preferred_element_type=jnp.float32)
    y = xl * dis
    z0_ref[...] = y[:, :128]
    z1_ref[...] = y[:, 128:]


def _stage3_body(a0_ref, a1_ref, dis_ref, b_ref, wlt_ref, bl_ref, out_ref):
    dis = dis_ref[...]
    h0 = jnp.maximum(a0_ref[...] * dis + b_ref[0, :128], 0.0)
    h1 = jnp.maximum(a1_ref[...] * dis + b_ref[0, 128:], 0.0)
    o = jnp.dot(h0, wlt_ref[0:128, :], preferred_element_type=jnp.float32)
    o = o + jnp.dot(h1, wlt_ref[128:256, :], preferred_element_type=jnp.float32)
    out_ref[...] = o + bl_ref[0, :]


def kernel(x, edge_index, W1, b1, W2, b2, Wl, bl):
    n, d_in = x.shape
    e = edge_index.shape[1]
    hid = W1.shape[0]
    ncls = Wl.shape[0]
    assert hid == 256 and d_in == 128 and n % NS == 0
    assert e % (NS * CH * SLAB_I) == 0

    cpt = e // (NS * CH)
    src3d = edge_index[0].reshape(NS, cpt, CH).astype(jnp.int32)
    dst3d = edge_index[1].reshape(NS, cpt, CH).astype(jnp.int32)
    zeros_n = jnp.zeros((n,), jnp.float32)
    ones_ch = jnp.ones((CH,), jnp.float32)

    deg = _make_deg_kernel(n, e)(dst3d, zeros_n, ones_ch).reshape(n, 1)

    blk = 1000
    grid = (n // blk,)

    y0, y1, dis = pl.pallas_call(
        _stage1_body,
        grid=grid,
        in_specs=[
            pl.BlockSpec((blk, 1), lambda i: (i, 0)),
            pl.BlockSpec((blk, d_in), lambda i: (i, 0)),
            pl.BlockSpec((d_in, hid), lambda i: (0, 0)),
        ],
        out_specs=[
            pl.BlockSpec((blk, 128), lambda i: (i, 0)),
            pl.BlockSpec((blk, 128), lambda i: (i, 0)),
            pl.BlockSpec((blk, 1), lambda i: (i, 0)),
        ],
        out_shape=[
            jax.ShapeDtypeStruct((n, 128), jnp.float32),
            jax.ShapeDtypeStruct((n, 128), jnp.float32),
            jax.ShapeDtypeStruct((n, 1), jnp.float32),
        ],
    )(deg, x, W1.T)

    agg = _make_agg_kernel(n, e, 128)
    a0, a1 = agg(y0, y1, src3d, dst3d)

    z0, z1 = pl.pallas_call(
        _stage2_body,
        grid=grid,
        in_specs=[
            pl.BlockSpec((blk, 128), lambda i: (i, 0)),
            pl.BlockSpec((blk, 128), lambda i: (i, 0)),
            pl.BlockSpec((blk, 1), lambda i: (i, 0)),
            pl.BlockSpec((1, hid), lambda i: (0, 0)),
            pl.BlockSpec((hid, hid), lambda i: (0, 0)),
        ],
        out_specs=[
            pl.BlockSpec((blk, 128), lambda i: (i, 0)),
            pl.BlockSpec((blk, 128), lambda i: (i, 0)),
        ],
        out_shape=[
            jax.ShapeDtypeStruct((n, 128), jnp.float32),
            jax.ShapeDtypeStruct((n, 128), jnp.float32),
        ],
    )(a0, a1, dis, b1.reshape(1, hid), W2.T)

    p0, p1 = agg(z0, z1, src3d, dst3d)

    out = pl.pallas_call(
        _stage3_body,
        grid=grid,
        in_specs=[
            pl.BlockSpec((blk, 128), lambda i: (i, 0)),
            pl.BlockSpec((blk, 128), lambda i: (i, 0)),
            pl.BlockSpec((blk, 1), lambda i: (i, 0)),
            pl.BlockSpec((1, hid), lambda i: (0, 0)),
            pl.BlockSpec((hid, ncls), lambda i: (0, 0)),
            pl.BlockSpec((1, ncls), lambda i: (0, 0)),
        ],
        out_specs=pl.BlockSpec((blk, ncls), lambda i: (i, 0)),
        out_shape=jax.ShapeDtypeStruct((n, ncls), jnp.float32),
    )(p0, p1, dis, b2.reshape(1, hid), Wl.T, bl.reshape(1, ncls))

    return out


# speedup vs baseline: 22.6753x; 1.0144x over previous
import functools
import jax
import jax.numpy as jnp
from jax import lax
from jax.experimental import pallas as pl
from jax.experimental.pallas import tpu as pltpu
from jax.experimental.pallas import tpu_sc as plsc

NC = 2
NS = 16
CH = 100
SLAB_I = 40
NB = 2


def _sc_mesh():
    return plsc.VectorSubcoreMesh(
        core_axis_name="c", subcore_axis_name="s", num_cores=NC, num_subcores=NS
    )


DEG_SLAB = 25


def _make_deg_kernel(n, e):
    cpt = e // (NS * CH)
    nsl = cpt // DEG_SLAB
    npc = nsl // NC

    @functools.partial(
        pl.kernel,
        out_type=jax.ShapeDtypeStruct((NC, n), jnp.float32),
        mesh=_sc_mesh(),
        scratch_types=[
            pltpu.VMEM_SHARED((n,), jnp.float32),
            pltpu.VMEM((DEG_SLAB, CH), jnp.int32),
            pltpu.VMEM((CH,), jnp.float32),
        ],
    )
    def deg_kernel(dst4d, zeros_hbm, ones_hbm, deg_out, acc, dstv, onesv):
        c = lax.axis_index("c")
        s = lax.axis_index("s")

        pltpu.sync_copy(ones_hbm, onesv)

        @pl.when(s == 0)
        def _():
            pltpu.sync_copy(zeros_hbm, acc)

        plsc.subcore_barrier()

        def slab_body(t, carry):
            pltpu.sync_copy(dst4d.at[s].at[c * npc + t], dstv)

            def chunk(j, carry2):
                pltpu.sync_copy(onesv, acc.at[dstv.at[j]], add=True)
                return carry2

            return lax.fori_loop(0, DEG_SLAB, chunk, carry)

        lax.fori_loop(0, npc, slab_body, 0)
        plsc.subcore_barrier()

        @pl.when(s == 0)
        def _():
            pltpu.sync_copy(acc, deg_out.at[c])

    return deg_kernel


def _make_agg_kernel(n, e, d):
    cpt = e // (NS * CH)
    nsl = cpt // SLAB_I
    slab = (n // NS) // 8 * 8
    tail = n - slab * NS

    @functools.partial(
        pl.kernel,
        out_type=(
            jax.ShapeDtypeStruct((n, d), jnp.float32),
            jax.ShapeDtypeStruct((n, d), jnp.float32),
        ),
        mesh=_sc_mesh(),
        scratch_types=[
            pltpu.VMEM_SHARED((n, d), jnp.float32),
            pltpu.VMEM((SLAB_I, CH), jnp.int32),
            pltpu.VMEM((SLAB_I, CH), jnp.int32),
            pltpu.VMEM((NB, CH, d), jnp.float32),
            pltpu.SemaphoreType.DMA,
            pltpu.SemaphoreType.DMA,
            pltpu.SemaphoreType.DMA,
            pltpu.SemaphoreType.DMA,
        ],
    )
    def agg_kernel(y0, y1, src3d, dst3d, o0, o1, acc, srcv, dstv, rows,
                   g0, g1, s0, s1):
        c = lax.axis_index("c")
        s = lax.axis_index("s")
        sem_g = [g0, g1]
        sem_s = [s0, s1]

        def half(y_hbm, out_hbm):
            pltpu.sync_copy(y_hbm.at[pl.ds(s * slab, slab)], acc.at[pl.ds(s * slab, slab)])

            @pl.when(s == NS - 1)
            def _():
                pltpu.sync_copy(y_hbm.at[pl.ds(slab * NS, tail)], acc.at[pl.ds(slab * NS, tail)])

            plsc.subcore_barrier()

            def slab_body(t, carry):
                pltpu.sync_copy(src3d.at[s].at[pl.ds(t * SLAB_I, SLAB_I)], srcv)
                pltpu.sync_copy(dst3d.at[s].at[pl.ds(t * SLAB_I, SLAB_I)], dstv)
                pltpu.async_copy(y_hbm.at[srcv.at[0]], rows.at[0], sem_g[0])

                def group(g, carry2):
                    for b in range(NB):
                        j = g * NB + b
                        bn = (b + 1) % NB
                        jn = j + 1

                        @pl.when(jn < SLAB_I)
                        def _():
                            @pl.when(jn >= NB)
                            def _():
                                pltpu.make_async_copy(
                                    rows.at[bn], acc.at[dstv.at[jn - NB]], sem_s[bn]
                                ).wait()

                            pltpu.async_copy(y_hbm.at[srcv.at[jn]], rows.at[bn], sem_g[bn])

                        pltpu.make_async_copy(
                            y_hbm.at[srcv.at[j]], rows.at[b], sem_g[b]
                        ).wait()
                        pltpu.async_copy(rows.at[b], acc.at[dstv.at[j]], sem_s[b], add=True)
                    return carry2

                lax.fori_loop(0, SLAB_I // NB, group, carry)
                for k in range(NB):
                    j = SLAB_I - NB + k
                    pltpu.make_async_copy(
                        rows.at[j % NB], acc.at[dstv.at[j]], sem_s[j % NB]
                    ).wait()
                return carry

            lax.fori_loop(0, nsl, slab_body, 0)
            plsc.subcore_barrier()
            pltpu.sync_copy(acc.at[pl.ds(s * slab, slab)], out_hbm.at[pl.ds(s * slab, slab)])

            @pl.when(s == NS - 1)
            def _():
                pltpu.sync_copy(acc.at[pl.ds(slab * NS, tail)], out_hbm.at[pl.ds(slab * NS, tail)])

        @pl.when(c == 0)
        def _():
            half(y0, o0)

        @pl.when(c == 1)
        def _():
            half(y1, o1)

    return agg_kernel


def _stage1_body(dega_ref, degb_ref, x_ref, w1t_ref, y0_ref, y1_ref, dis_ref):
    dis = lax.rsqrt(dega_ref[...] + degb_ref[...] + 1.0)
    xl = jnp.dot(x_ref[...], w1t_ref[...], preferred_element_type=jnp.float32)
    y = xl * dis
    y0_ref[...] = y[:, :128]
    y1_ref[...] = y[:, 128:]
    dis_ref[...] = dis


def _stage2_body(a0_ref, a1_ref, dis_ref, b_ref, w2t_ref, z0_ref, z1_ref):
    dis = dis_ref[...]
    h0 = jnp.maximum(a0_ref[...] * dis + b_ref[0, :128], 0.0)
    h1 = jnp.maximum(a1_ref[...] * dis + b_ref[0, 128:], 0.0)
    xl = jnp.dot(h0, w2t_ref[0:128, :], preferred_element_type=jnp.float32)
    xl = xl + jnp.dot(h1, w2t_ref[128:256, :], preferred_element_type=jnp.float32)
    y = xl * dis
    z0_ref[...] = y[:, :128]
    z1_ref[...] = y[:, 128:]


def _stage3_body(a0_ref, a1_ref, dis_ref, b_ref, wlt_ref, bl_ref, out_ref):
    dis = dis_ref[...]
    h0 = jnp.maximum(a0_ref[...] * dis + b_ref[0, :128], 0.0)
    h1 = jnp.maximum(a1_ref[...] * dis + b_ref[0, 128:], 0.0)
    o = jnp.dot(h0, wlt_ref[0:128, :], preferred_element_type=jnp.float32)
    o = o + jnp.dot(h1, wlt_ref[128:256, :], preferred_element_type=jnp.float32)
    out_ref[...] = o + bl_ref[0, :]


def kernel(x, edge_index, W1, b1, W2, b2, Wl, bl):
    n, d_in = x.shape
    e = edge_index.shape[1]
    hid = W1.shape[0]
    ncls = Wl.shape[0]
    assert hid == 256 and d_in == 128 and n % NS == 0
    assert e % (NS * CH * SLAB_I) == 0

    cpt = e // (NS * CH)
    src3d = edge_index[0].reshape(NS, cpt, CH).astype(jnp.int32)
    dst3d = edge_index[1].reshape(NS, cpt, CH).astype(jnp.int32)
    dst4d = edge_index[1].reshape(NS, cpt // DEG_SLAB, DEG_SLAB, CH).astype(jnp.int32)
    zeros_n = jnp.zeros((n,), jnp.float32)
    ones_ch = jnp.ones((CH,), jnp.float32)

    degp = _make_deg_kernel(n, e)(dst4d, zeros_n, ones_ch).reshape(NC * n, 1)

    blk = 1000
    grid = (n // blk,)
    nblk = n // blk

    y0, y1, dis = pl.pallas_call(
        _stage1_body,
        grid=grid,
        in_specs=[
            pl.BlockSpec((blk, 1), lambda i: (i, 0)),
            pl.BlockSpec((blk, 1), lambda i: (nblk + i, 0)),
            pl.BlockSpec((blk, d_in), lambda i: (i, 0)),
            pl.BlockSpec((d_in, hid), lambda i: (0, 0)),
        ],
        out_specs=[
            pl.BlockSpec((blk, 128), lambda i: (i, 0)),
            pl.BlockSpec((blk, 128), lambda i: (i, 0)),
            pl.BlockSpec((blk, 1), lambda i: (i, 0)),
        ],
        out_shape=[
            jax.ShapeDtypeStruct((n, 128), jnp.float32),
            jax.ShapeDtypeStruct((n, 128), jnp.float32),
            jax.ShapeDtypeStruct((n, 1), jnp.float32),
        ],
    )(degp, degp, x, W1.T)

    agg = _make_agg_kernel(n, e, 128)
    a0, a1 = agg(y0, y1, src3d, dst3d)

    z0, z1 = pl.pallas_call(
        _stage2_body,
        grid=grid,
        in_specs=[
            pl.BlockSpec((blk, 128), lambda i: (i, 0)),
            pl.BlockSpec((blk, 128), lambda i: (i, 0)),
            pl.BlockSpec((blk, 1), lambda i: (i, 0)),
            pl.BlockSpec((1, hid), lambda i: (0, 0)),
            pl.BlockSpec((hid, hid), lambda i: (0, 0)),
        ],
        out_specs=[
            pl.BlockSpec((blk, 128), lambda i: (i, 0)),
            pl.BlockSpec((blk, 128), lambda i: (i, 0)),
        ],
        out_shape=[
            jax.ShapeDtypeStruct((n, 128), jnp.float32),
            jax.ShapeDtypeStruct((n, 128), jnp.float32),
        ],
    )(a0, a1, dis, b1.reshape(1, hid), W2.T)

    p0, p1 = agg(z0, z1, src3d, dst3d)

    out = pl.pallas_call(
        _stage3_body,
        grid=grid,
        in_specs=[
            pl.BlockSpec((blk, 128), lambda i: (i, 0)),
            pl.BlockSpec((blk, 128), lambda i: (i, 0)),
            pl.BlockSpec((blk, 1), lambda i: (i, 0)),
            pl.BlockSpec((1, hid), lambda i: (0, 0)),
            pl.BlockSpec((hid, ncls), lambda i: (0, 0)),
            pl.BlockSpec((1, ncls), lambda i: (0, 0)),
        ],
        out_specs=pl.BlockSpec((blk, ncls), lambda i: (i, 0)),
        out_shape=jax.ShapeDtypeStruct((n, ncls), jnp.float32),
    )(p0, p1, dis, b2.reshape(1, hid), Wl.T, bl.reshape(1, ncls))

    return out


# speedup vs baseline: 23.6932x; 1.0449x over previous
import functools
import jax
import jax.numpy as jnp
from jax import lax
from jax.experimental import pallas as pl
from jax.experimental.pallas import tpu as pltpu
from jax.experimental.pallas import tpu_sc as plsc

NC = 2
NS = 16
CH = 125
SLAB_I = 40
NB = 2
DCH = 100


def _sc_mesh():
    return plsc.VectorSubcoreMesh(
        core_axis_name="c", subcore_axis_name="s", num_cores=NC, num_subcores=NS
    )


DEG_SLAB = 25


def _make_deg_kernel(n, e):
    cpt = e // (NS * DCH)
    nsl = cpt // DEG_SLAB
    npc = nsl // NC

    @functools.partial(
        pl.kernel,
        out_type=jax.ShapeDtypeStruct((NC, n), jnp.float32),
        mesh=_sc_mesh(),
        scratch_types=[
            pltpu.VMEM_SHARED((n,), jnp.float32),
            pltpu.VMEM((DEG_SLAB, DCH), jnp.int32),
            pltpu.VMEM((DCH,), jnp.float32),
        ],
    )
    def deg_kernel(dst4d, zeros_hbm, ones_hbm, deg_out, acc, dstv, onesv):
        c = lax.axis_index("c")
        s = lax.axis_index("s")

        pltpu.sync_copy(ones_hbm, onesv)

        @pl.when(s == 0)
        def _():
            pltpu.sync_copy(zeros_hbm, acc)

        plsc.subcore_barrier()

        def slab_body(t, carry):
            pltpu.sync_copy(dst4d.at[s].at[c * npc + t], dstv)

            def chunk(j, carry2):
                pltpu.sync_copy(onesv, acc.at[dstv.at[j]], add=True)
                return carry2

            return lax.fori_loop(0, DEG_SLAB, chunk, carry)

        lax.fori_loop(0, npc, slab_body, 0)
        plsc.subcore_barrier()

        @pl.when(s == 0)
        def _():
            pltpu.sync_copy(acc, deg_out.at[c])

    return deg_kernel


def _make_agg_kernel(n, e, d):
    cpt = e // (NS * CH)
    nsl = cpt // SLAB_I
    slab = (n // NS) // 8 * 8
    tail = n - slab * NS

    @functools.partial(
        pl.kernel,
        out_type=(
            jax.ShapeDtypeStruct((n, d), jnp.float32),
            jax.ShapeDtypeStruct((n, d), jnp.float32),
        ),
        mesh=_sc_mesh(),
        scratch_types=[
            pltpu.VMEM_SHARED((n, d), jnp.float32),
            pltpu.VMEM((SLAB_I, CH), jnp.int32),
            pltpu.VMEM((SLAB_I, CH), jnp.int32),
            pltpu.VMEM((NB, CH, d), jnp.float32),
            pltpu.SemaphoreType.DMA,
            pltpu.SemaphoreType.DMA,
            pltpu.SemaphoreType.DMA,
            pltpu.SemaphoreType.DMA,
        ],
    )
    def agg_kernel(y0, y1, src3d, dst3d, o0, o1, acc, srcv, dstv, rows,
                   g0, g1, s0, s1):
        c = lax.axis_index("c")
        s = lax.axis_index("s")
        sem_g = [g0, g1]
        sem_s = [s0, s1]

        def half(y_hbm, out_hbm):
            pltpu.sync_copy(y_hbm.at[pl.ds(s * slab, slab)], acc.at[pl.ds(s * slab, slab)])

            @pl.when(s == NS - 1)
            def _():
                pltpu.sync_copy(y_hbm.at[pl.ds(slab * NS, tail)], acc.at[pl.ds(slab * NS, tail)])

            plsc.subcore_barrier()

            def slab_body(t, carry):
                pltpu.sync_copy(src3d.at[s].at[pl.ds(t * SLAB_I, SLAB_I)], srcv)
                pltpu.sync_copy(dst3d.at[s].at[pl.ds(t * SLAB_I, SLAB_I)], dstv)
                pltpu.async_copy(y_hbm.at[srcv.at[0]], rows.at[0], sem_g[0])

                def group(g, carry2):
                    for b in range(NB):
                        j = g * NB + b
                        bn = (b + 1) % NB
                        jn = j + 1

                        @pl.when(jn < SLAB_I)
                        def _():
                            @pl.when(jn >= NB)
                            def _():
                                pltpu.make_async_copy(
                                    rows.at[bn], acc.at[dstv.at[jn - NB]], sem_s[bn]
                                ).wait()

                            pltpu.async_copy(y_hbm.at[srcv.at[jn]], rows.at[bn], sem_g[bn])

                        pltpu.make_async_copy(
                            y_hbm.at[srcv.at[j]], rows.at[b], sem_g[b]
                        ).wait()
                        pltpu.async_copy(rows.at[b], acc.at[dstv.at[j]], sem_s[b], add=True)
                    return carry2

                lax.fori_loop(0, SLAB_I // NB, group, carry)
                for k in range(NB):
                    j = SLAB_I - NB + k
                    pltpu.make_async_copy(
                        rows.at[j % NB], acc.at[dstv.at[j]], sem_s[j % NB]
                    ).wait()
                return carry

            lax.fori_loop(0, nsl, slab_body, 0)
            plsc.subcore_barrier()
            pltpu.sync_copy(acc.at[pl.ds(s * slab, slab)], out_hbm.at[pl.ds(s * slab, slab)])

            @pl.when(s == NS - 1)
            def _():
                pltpu.sync_copy(acc.at[pl.ds(slab * NS, tail)], out_hbm.at[pl.ds(slab * NS, tail)])

        @pl.when(c == 0)
        def _():
            half(y0, o0)

        @pl.when(c == 1)
        def _():
            half(y1, o1)

    return agg_kernel


def _stage1_body(dega_ref, degb_ref, x_ref, w1t_ref, y0_ref, y1_ref, dis_ref):
    dis = lax.rsqrt(dega_ref[...] + degb_ref[...] + 1.0)
    xl = jnp.dot(x_ref[...], w1t_ref[...], preferred_element_type=jnp.float32)
    y = xl * dis
    y0_ref[...] = y[:, :128]
    y1_ref[...] = y[:, 128:]
    dis_ref[...] = dis


def _stage2_body(a0_ref, a1_ref, dis_ref, b_ref, w2t_ref, z0_ref, z1_ref):
    dis = dis_ref[...]
    h0 = jnp.maximum(a0_ref[...] * dis + b_ref[0, :128], 0.0)
    h1 = jnp.maximum(a1_ref[...] * dis + b_ref[0, 128:], 0.0)
    xl = jnp.dot(h0, w2t_ref[0:128, :], preferred_element_type=jnp.float32)
    xl = xl + jnp.dot(h1, w2t_ref[128:256, :], preferred_element_type=jnp.float32)
    y = xl * dis
    z0_ref[...] = y[:, :128]
    z1_ref[...] = y[:, 128:]


def _stage3_body(a0_ref, a1_ref, dis_ref, b_ref, wlt_ref, bl_ref, out_ref):
    dis = dis_ref[...]
    h0 = jnp.maximum(a0_ref[...] * dis + b_ref[0, :128], 0.0)
    h1 = jnp.maximum(a1_ref[...] * dis + b_ref[0, 128:], 0.0)
    o = jnp.dot(h0, wlt_ref[0:128, :], preferred_element_type=jnp.float32)
    o = o + jnp.dot(h1, wlt_ref[128:256, :], preferred_element_type=jnp.float32)
    out_ref[...] = o + bl_ref[0, :]


def kernel(x, edge_index, W1, b1, W2, b2, Wl, bl):
    n, d_in = x.shape
    e = edge_index.shape[1]
    hid = W1.shape[0]
    ncls = Wl.shape[0]
    assert hid == 256 and d_in == 128 and n % NS == 0
    assert e % (NS * CH * SLAB_I) == 0

    cpt = e // (NS * CH)
    src3d = edge_index[0].reshape(NS, cpt, CH).astype(jnp.int32)
    dst3d = edge_index[1].reshape(NS, cpt, CH).astype(jnp.int32)
    dcpt = e // (NS * DCH)
    dst4d = edge_index[1].reshape(NS, dcpt // DEG_SLAB, DEG_SLAB, DCH).astype(jnp.int32)
    zeros_n = jnp.zeros((n,), jnp.float32)
    ones_ch = jnp.ones((DCH,), jnp.float32)

    degp = _make_deg_kernel(n, e)(dst4d, zeros_n, ones_ch).reshape(NC * n, 1)

    blk = 1000
    grid = (n // blk,)
    nblk = n // blk

    y0, y1, dis = pl.pallas_call(
        _stage1_body,
        grid=grid,
        in_specs=[
            pl.BlockSpec((blk, 1), lambda i: (i, 0)),
            pl.BlockSpec((blk, 1), lambda i: (nblk + i, 0)),
            pl.BlockSpec((blk, d_in), lambda i: (i, 0)),
            pl.BlockSpec((d_in, hid), lambda i: (0, 0)),
        ],
        out_specs=[
            pl.BlockSpec((blk, 128), lambda i: (i, 0)),
            pl.BlockSpec((blk, 128), lambda i: (i, 0)),
            pl.BlockSpec((blk, 1), lambda i: (i, 0)),
        ],
        out_shape=[
            jax.ShapeDtypeStruct((n, 128), jnp.float32),
            jax.ShapeDtypeStruct((n, 128), jnp.float32),
            jax.ShapeDtypeStruct((n, 1), jnp.float32),
        ],
    )(degp, degp, x, W1.T)

    agg = _make_agg_kernel(n, e, 128)
    a0, a1 = agg(y0, y1, src3d, dst3d)

    z0, z1 = pl.pallas_call(
        _stage2_body,
        grid=grid,
        in_specs=[
            pl.BlockSpec((blk, 128), lambda i: (i, 0)),
            pl.BlockSpec((blk, 128), lambda i: (i, 0)),
            pl.BlockSpec((blk, 1), lambda i: (i, 0)),
            pl.BlockSpec((1, hid), lambda i: (0, 0)),
            pl.BlockSpec((hid, hid), lambda i: (0, 0)),
        ],
        out_specs=[
            pl.BlockSpec((blk, 128), lambda i: (i, 0)),
            pl.BlockSpec((blk, 128), lambda i: (i, 0)),
        ],
        out_shape=[
            jax.ShapeDtypeStruct((n, 128), jnp.float32),
            jax.ShapeDtypeStruct((n, 128), jnp.float32),
        ],
    )(a0, a1, dis, b1.reshape(1, hid), W2.T)

    p0, p1 = agg(z0, z1, src3d, dst3d)

    out = pl.pallas_call(
        _stage3_body,
        grid=grid,
        in_specs=[
            pl.BlockSpec((blk, 128), lambda i: (i, 0)),
            pl.BlockSpec((blk, 128), lambda i: (i, 0)),
            pl.BlockSpec((blk, 1), lambda i: (i, 0)),
            pl.BlockSpec((1, hid), lambda i: (0, 0)),
            pl.BlockSpec((hid, ncls), lambda i: (0, 0)),
            pl.BlockSpec((1, ncls), lambda i: (0, 0)),
        ],
        out_specs=pl.BlockSpec((blk, ncls), lambda i: (i, 0)),
        out_shape=jax.ShapeDtypeStruct((n, ncls), jnp.float32),
    )(p0, p1, dis, b2.reshape(1, hid), Wl.T, bl.reshape(1, ncls))

    return out


# speedup vs baseline: 23.9008x; 1.0088x over previous
import functools
import jax
import jax.numpy as jnp
from jax import lax
from jax.experimental import pallas as pl
from jax.experimental.pallas import tpu as pltpu
from jax.experimental.pallas import tpu_sc as plsc

NC = 2
NS = 16
CH = 125
SLAB_I = 40
NB = 2
DCH = 125


def _sc_mesh():
    return plsc.VectorSubcoreMesh(
        core_axis_name="c", subcore_axis_name="s", num_cores=NC, num_subcores=NS
    )


DEG_SLAB = 20


def _make_deg_kernel(n, e):
    cpt = e // (NS * DCH)
    nsl = cpt // DEG_SLAB
    npc = nsl // NC

    @functools.partial(
        pl.kernel,
        out_type=jax.ShapeDtypeStruct((NC, n), jnp.float32),
        mesh=_sc_mesh(),
        scratch_types=[
            pltpu.VMEM_SHARED((n,), jnp.float32),
            pltpu.VMEM((DEG_SLAB, DCH), jnp.int32),
            pltpu.VMEM((DCH,), jnp.float32),
        ],
    )
    def deg_kernel(dst4d, zeros_hbm, ones_hbm, deg_out, acc, dstv, onesv):
        c = lax.axis_index("c")
        s = lax.axis_index("s")

        pltpu.sync_copy(ones_hbm, onesv)

        @pl.when(s == 0)
        def _():
            pltpu.sync_copy(zeros_hbm, acc)

        plsc.subcore_barrier()

        def slab_body(t, carry):
            pltpu.sync_copy(dst4d.at[s].at[c * npc + t], dstv)

            def chunk(j, carry2):
                pltpu.sync_copy(onesv, acc.at[dstv.at[j]], add=True)
                return carry2

            return lax.fori_loop(0, DEG_SLAB, chunk, carry)

        lax.fori_loop(0, npc, slab_body, 0)
        plsc.subcore_barrier()

        @pl.when(s == 0)
        def _():
            pltpu.sync_copy(acc, deg_out.at[c])

    return deg_kernel


def _make_agg_kernel(n, e, d):
    cpt = e // (NS * CH)
    nsl = cpt // SLAB_I
    slab = (n // NS) // 8 * 8
    tail = n - slab * NS

    @functools.partial(
        pl.kernel,
        out_type=(
            jax.ShapeDtypeStruct((n, d), jnp.float32),
            jax.ShapeDtypeStruct((n, d), jnp.float32),
        ),
        mesh=_sc_mesh(),
        scratch_types=[
            pltpu.VMEM_SHARED((n, d), jnp.float32),
            pltpu.VMEM((SLAB_I, CH), jnp.int32),
            pltpu.VMEM((SLAB_I, CH), jnp.int32),
            pltpu.VMEM((NB, CH, d), jnp.float32),
            pltpu.SemaphoreType.DMA,
            pltpu.SemaphoreType.DMA,
            pltpu.SemaphoreType.DMA,
            pltpu.SemaphoreType.DMA,
        ],
    )
    def agg_kernel(y0, y1, src3d, dst3d, o0, o1, acc, srcv, dstv, rows,
                   g0, g1, s0, s1):
        c = lax.axis_index("c")
        s = lax.axis_index("s")
        sem_g = [g0, g1]
        sem_s = [s0, s1]

        def half(y_hbm, out_hbm):
            pltpu.sync_copy(y_hbm.at[pl.ds(s * slab, slab)], acc.at[pl.ds(s * slab, slab)])

            @pl.when(s == NS - 1)
            def _():
                pltpu.sync_copy(y_hbm.at[pl.ds(slab * NS, tail)], acc.at[pl.ds(slab * NS, tail)])

            plsc.subcore_barrier()

            def slab_body(t, carry):
                pltpu.sync_copy(src3d.at[s].at[pl.ds(t * SLAB_I, SLAB_I)], srcv)
                pltpu.sync_copy(dst3d.at[s].at[pl.ds(t * SLAB_I, SLAB_I)], dstv)
                pltpu.async_copy(y_hbm.at[srcv.at[0]], rows.at[0], sem_g[0])

                def group(g, carry2):
                    for b in range(NB):
                        j = g * NB + b
                        bn = (b + 1) % NB
                        jn = j + 1

                        @pl.when(jn < SLAB_I)
                        def _():
                            @pl.when(jn >= NB)
                            def _():
                                pltpu.make_async_copy(
                                    rows.at[bn], acc.at[dstv.at[jn - NB]], sem_s[bn]
                                ).wait()

                            pltpu.async_copy(y_hbm.at[srcv.at[jn]], rows.at[bn], sem_g[bn])

                        pltpu.make_async_copy(
                            y_hbm.at[srcv.at[j]], rows.at[b], sem_g[b]
                        ).wait()
                        pltpu.async_copy(rows.at[b], acc.at[dstv.at[j]], sem_s[b], add=True)
                    return carry2

                lax.fori_loop(0, SLAB_I // NB, group, carry)
                for k in range(NB):
                    j = SLAB_I - NB + k
                    pltpu.make_async_copy(
                        rows.at[j % NB], acc.at[dstv.at[j]], sem_s[j % NB]
                    ).wait()
                return carry

            lax.fori_loop(0, nsl, slab_body, 0)
            plsc.subcore_barrier()
            pltpu.sync_copy(acc.at[pl.ds(s * slab, slab)], out_hbm.at[pl.ds(s * slab, slab)])

            @pl.when(s == NS - 1)
            def _():
                pltpu.sync_copy(acc.at[pl.ds(slab * NS, tail)], out_hbm.at[pl.ds(slab * NS, tail)])

        @pl.when(c == 0)
        def _():
            half(y0, o0)

        @pl.when(c == 1)
        def _():
            half(y1, o1)

    return agg_kernel


def _stage1_body(dega_ref, degb_ref, x_ref, w1t_ref, y0_ref, y1_ref, dis_ref):
    dis = lax.rsqrt(dega_ref[...] + degb_ref[...] + 1.0)
    xl = jnp.dot(x_ref[...], w1t_ref[...], preferred_element_type=jnp.float32)
    y = xl * dis
    y0_ref[...] = y[:, :128]
    y1_ref[...] = y[:, 128:]
    dis_ref[...] = dis


def _stage2_body(a0_ref, a1_ref, dis_ref, b_ref, w2t_ref, z0_ref, z1_ref):
    dis = dis_ref[...]
    h0 = jnp.maximum(a0_ref[...] * dis + b_ref[0, :128], 0.0)
    h1 = jnp.maximum(a1_ref[...] * dis + b_ref[0, 128:], 0.0)
    xl = jnp.dot(h0, w2t_ref[0:128, :], preferred_element_type=jnp.float32)
    xl = xl + jnp.dot(h1, w2t_ref[128:256, :], preferred_element_type=jnp.float32)
    y = xl * dis
    z0_ref[...] = y[:, :128]
    z1_ref[...] = y[:, 128:]


def _stage3_body(a0_ref, a1_ref, dis_ref, b_ref, wlt_ref, bl_ref, out_ref):
    dis = dis_ref[...]
    h0 = jnp.maximum(a0_ref[...] * dis + b_ref[0, :128], 0.0)
    h1 = jnp.maximum(a1_ref[...] * dis + b_ref[0, 128:], 0.0)
    o = jnp.dot(h0, wlt_ref[0:128, :], preferred_element_type=jnp.float32)
    o = o + jnp.dot(h1, wlt_ref[128:256, :], preferred_element_type=jnp.float32)
    out_ref[...] = o + bl_ref[0, :]


def kernel(x, edge_index, W1, b1, W2, b2, Wl, bl):
    n, d_in = x.shape
    e = edge_index.shape[1]
    hid = W1.shape[0]
    ncls = Wl.shape[0]
    assert hid == 256 and d_in == 128 and n % NS == 0
    assert e % (NS * CH * SLAB_I) == 0

    cpt = e // (NS * CH)
    src3d = edge_index[0].reshape(NS, cpt, CH).astype(jnp.int32)
    dst3d = edge_index[1].reshape(NS, cpt, CH).astype(jnp.int32)
    dcpt = e // (NS * DCH)
    dst4d = edge_index[1].reshape(NS, dcpt // DEG_SLAB, DEG_SLAB, DCH).astype(jnp.int32)
    zeros_n = jnp.zeros((n,), jnp.float32)
    ones_ch = jnp.ones((DCH,), jnp.float32)

    degp = _make_deg_kernel(n, e)(dst4d, zeros_n, ones_ch).reshape(NC * n, 1)

    blk = 2000
    grid = (n // blk,)
    nblk = n // blk

    y0, y1, dis = pl.pallas_call(
        _stage1_body,
        grid=grid,
        in_specs=[
            pl.BlockSpec((blk, 1), lambda i: (i, 0)),
            pl.BlockSpec((blk, 1), lambda i: (nblk + i, 0)),
            pl.BlockSpec((blk, d_in), lambda i: (i, 0)),
            pl.BlockSpec((d_in, hid), lambda i: (0, 0)),
        ],
        out_specs=[
            pl.BlockSpec((blk, 128), lambda i: (i, 0)),
            pl.BlockSpec((blk, 128), lambda i: (i, 0)),
            pl.BlockSpec((blk, 1), lambda i: (i, 0)),
        ],
        out_shape=[
            jax.ShapeDtypeStruct((n, 128), jnp.float32),
            jax.ShapeDtypeStruct((n, 128), jnp.float32),
            jax.ShapeDtypeStruct((n, 1), jnp.float32),
        ],
    )(degp, degp, x, W1.T)

    agg = _make_agg_kernel(n, e, 128)
    a0, a1 = agg(y0, y1, src3d, dst3d)

    z0, z1 = pl.pallas_call(
        _stage2_body,
        grid=grid,
        in_specs=[
            pl.BlockSpec((blk, 128), lambda i: (i, 0)),
            pl.BlockSpec((blk, 128), lambda i: (i, 0)),
            pl.BlockSpec((blk, 1), lambda i: (i, 0)),
            pl.BlockSpec((1, hid), lambda i: (0, 0)),
            pl.BlockSpec((hid, hid), lambda i: (0, 0)),
        ],
        out_specs=[
            pl.BlockSpec((blk, 128), lambda i: (i, 0)),
            pl.BlockSpec((blk, 128), lambda i: (i, 0)),
        ],
        out_shape=[
            jax.ShapeDtypeStruct((n, 128), jnp.float32),
            jax.ShapeDtypeStruct((n, 128), jnp.float32),
        ],
    )(a0, a1, dis, b1.reshape(1, hid), W2.T)

    p0, p1 = agg(z0, z1, src3d, dst3d)

    out = pl.pallas_call(
        _stage3_body,
        grid=grid,
        in_specs=[
            pl.BlockSpec((blk, 128), lambda i: (i, 0)),
            pl.BlockSpec((blk, 128), lambda i: (i, 0)),
            pl.BlockSpec((blk, 1), lambda i: (i, 0)),
            pl.BlockSpec((1, hid), lambda i: (0, 0)),
            pl.BlockSpec((hid, ncls), lambda i: (0, 0)),
            pl.BlockSpec((1, ncls), lambda i: (0, 0)),
        ],
        out_specs=pl.BlockSpec((blk, ncls), lambda i: (i, 0)),
        out_shape=jax.ShapeDtypeStruct((n, ncls), jnp.float32),
    )(p0, p1, dis, b2.reshape(1, hid), Wl.T, bl.reshape(1, ncls))

    return out
